# baseline (speedup 1.0000x reference)
import functools

import jax
import jax.numpy as jnp
from jax import lax
from jax.experimental import pallas as pl
from jax.experimental.pallas import tpu as pltpu
from jax.experimental.pallas import tpu_sc as plsc

_V = 100000
_D = 32
_B = 16384
_W = 3125
_WIN = 3328
_WS = 3360
_BMAX = 96640
_TAIL0 = _V - 32
_PH = _B // 2
_IH = 4096
_PB = 512
_IMROWS = _PH + 8
_NSLOT = 48
_FLUSH_AT = 32


def _build():
    mesh = plsc.VectorSubcoreMesh(core_axis_name="c", subcore_axis_name="s")

    @functools.partial(
        pl.kernel,
        mesh=mesh,
        out_type=(
            jax.ShapeDtypeStruct((_D, _B), jnp.float32),
            jax.ShapeDtypeStruct((2 * _IMROWS, 128), jnp.float32),
        ),
        compiler_params=pltpu.CompilerParams(
            use_tc_tiling_on_sc=True, needs_layout_passes=False),
        scratch_types=[
            pltpu.VMEM((_D, _WS), jnp.float32),
            pltpu.VMEM((_IH,), jnp.int32),
            pltpu.VMEM((_D, 32), jnp.float32),
            pltpu.VMEM((_NSLOT,), jnp.int32),
            pltpu.VMEM((_NSLOT,), jnp.int32),
            pltpu.VMEM((_NSLOT, 128), jnp.float32),
            pltpu.VMEM((_D, 128), jnp.float32),
        ],
    )
    def gather_kernel(tbl_t, idx_hbm, tail_t, out_t, image, stage, idxb,
                      tailb, hits_c, hits_p, records, outbuf):
        core = lax.axis_index("c")
        tile = lax.axis_index("s")
        lanes = lax.iota(jnp.int32, 16)
        im0 = core * _IMROWS
        dump = im0 + _PH

        def reset_hits():
            zeros = jnp.zeros((16,), jnp.int32)
            dumpv = jnp.zeros((16,), jnp.int32) + dump
            for q in range(_NSLOT // 16):
                hits_c[pl.ds(16 * q, 16)] = zeros
                hits_p[pl.ds(16 * q, 16)] = dumpv

        def flush():
            for g in range(_NSLOT // 16):
                cvec = hits_c[pl.ds(16 * g, 16)]
                slot = lanes + (16 * g)
                for d in range(_D):
                    dvec = jnp.full((16,), d, jnp.int32)
                    vals = plsc.load_gather(stage, [dvec, cvec])
                    plsc.store_scatter(records, [slot, dvec], vals)
            pltpu.sync_copy(records, image.at[hits_p])
            reset_hits()

        pltpu.sync_copy(tail_t, tailb)
        reset_hits()

        for gen in range(2):
            r0 = (gen * 16 + tile) * _W
            base = jnp.minimum((r0 // 128) * 128, _BMAX)
            pltpu.sync_copy(tbl_t.at[:, pl.ds(base, _WIN)],
                            stage.at[:, pl.ds(0, _WIN)])
            if gen == 1:
                for d in range(_D):
                    for q in range(2):
                        stage[d, pl.ds(_WIN + 16 * q, 16)] = (
                            tailb[d, pl.ds(16 * q, 16)])

            for half in range(2):
                pltpu.sync_copy(
                    idx_hbm.at[pl.ds(core * _PH + half * _IH, _IH)], idxb)

                def scan_chunk(j, cnt, r0=r0, base=base, half=half):
                    iv = idxb[pl.ds(16 * j, 16)]
                    m = (iv >= r0) & (iv < r0 + _W)
                    crel = jnp.where(iv >= _TAIL0,
                                     _WIN + iv - _TAIL0, iv - base)
                    pm = plsc.cumsum(m.astype(jnp.int32))
                    dest = cnt + pm - 1
                    pvec = im0 + half * _IH + 16 * j + lanes
                    plsc.store_scatter(hits_c, [dest], crel, mask=m)
                    plsc.store_scatter(hits_p, [dest], pvec, mask=m)
                    cnt2 = cnt + jnp.sum(m.astype(jnp.int32))
                    full = cnt2 >= _FLUSH_AT

                    @pl.when(full)
                    def _():
                        flush()

                    return jnp.where(full, 0, cnt2)

                lax.fori_loop(0, _IH // 16, scan_chunk, jnp.int32(0))
                flush()

        plsc.subcore_barrier()

        def write_block(b, carry):
            row0 = tile * _PB + 128 * b
            for h in range(128 // _NSLOT + 1):
                n = min(_NSLOT, 128 - h * _NSLOT)
                pltpu.sync_copy(image.at[pl.ds(im0 + row0 + h * _NSLOT, n)],
                                records.at[pl.ds(0, n)])
                for g in range(-(-n // 16)):
                    pvec = lanes + (16 * g)
                    for d in range(_D):
                        dvec = jnp.full((16,), d, jnp.int32)
                        vals = plsc.load_gather(records, [pvec, dvec])
                        outbuf[d, pl.ds(h * _NSLOT + 16 * g, 16)] = vals
            pltpu.sync_copy(
                outbuf, out_t.at[:, pl.ds(core * _PH + row0, 128)])
            return carry

        lax.fori_loop(0, _PB // 128, write_block, jnp.int32(0))

    return gather_kernel


def kernel(theme_ids, table):
    gather_kernel = _build()
    out_t, _ = gather_kernel(table.T, theme_ids.astype(jnp.int32),
                             table[_TAIL0:, :].T)
    return out_t.T
